# Initial kernel scaffold; baseline (speedup 1.0000x reference)
import functools

import jax
import jax.numpy as jnp
from jax import lax
from jax.experimental import pallas as pl
from jax.experimental.pallas import tpu as pltpu
from jax.experimental.pallas import tpu_sc as plsc

_NC = 2
_NS = 16
_NW = _NC * _NS

_D = 64
_KP = 16

_CHUNK = 512
_IDXW = 128
_NFIRE = _CHUNK // _IDXW


def _proj_body(tab_ref, w_ref, b_ref, out_ref):
    out_ref[...] = (
        jnp.dot(tab_ref[...], w_ref[...], preferred_element_type=jnp.float32)
        + b_ref[...]
    )


def _project_table(embed_table, w_pad, b_pad):
    V, D = embed_table.shape
    blk = 4000
    return pl.pallas_call(
        _proj_body,
        grid=(V // blk,),
        in_specs=[
            pl.BlockSpec((blk, D), lambda i: (i, 0)),
            pl.BlockSpec((D, _KP), lambda i: (0, 0)),
            pl.BlockSpec((1, _KP), lambda i: (0, 0)),
        ],
        out_specs=pl.BlockSpec((blk, _KP), lambda i: (i, 0)),
        out_shape=jax.ShapeDtypeStruct((V, _KP), jnp.float32),
    )(embed_table, w_pad, b_pad)


def _sc_gather(embed_table, proj_table, ids2d):
    n_rows = ids2d.shape[0] * ids2d.shape[1]
    per_w = n_rows // _NW
    n_chunks = per_w // _CHUNK
    mesh = plsc.VectorSubcoreMesh(
        core_axis_name="c", subcore_axis_name="s",
        num_cores=_NC, num_subcores=_NS,
    )

    @functools.partial(
        pl.kernel,
        mesh=mesh,
        out_type=(
            jax.ShapeDtypeStruct((n_rows, _D), jnp.float32),
            jax.ShapeDtypeStruct((n_rows, _KP), jnp.float32),
        ),
        scratch_types=[
            pltpu.VMEM((_NFIRE, _IDXW), jnp.int32),
            pltpu.VMEM((_CHUNK, _D), jnp.float32),
            pltpu.VMEM((_CHUNK, _KP), jnp.float32),
            pltpu.SemaphoreType.DMA,
        ],
    )
    def gather_kernel(tab_hbm, ptab_hbm, ids_hbm, emb_out, log_out,
                      idx_v, emb_v, log_v, sem):
        wid = lax.axis_index("s") * _NC + lax.axis_index("c")
        w_base = wid * per_w

        def body(i, _):
            base = w_base + i * _CHUNK
            pltpu.sync_copy(ids_hbm.at[pl.ds(base // _IDXW, _NFIRE)], idx_v)
            copies = []
            for j in range(_NFIRE):
                copies.append(pltpu.async_copy(
                    tab_hbm.at[idx_v.at[j]],
                    emb_v.at[pl.ds(j * _IDXW, _IDXW)], sem))
                copies.append(pltpu.async_copy(
                    ptab_hbm.at[idx_v.at[j]],
                    log_v.at[pl.ds(j * _IDXW, _IDXW)], sem))
            for c in copies:
                c.wait()
            pltpu.sync_copy(emb_v, emb_out.at[pl.ds(base, _CHUNK)])
            pltpu.sync_copy(log_v, log_out.at[pl.ds(base, _CHUNK)])
            return ()

        lax.fori_loop(0, n_chunks, body, ())

    return gather_kernel(embed_table, proj_table, ids2d)


def kernel(text_ids, embed_table, dur_w, dur_b):
    B, T = text_ids.shape
    w_pad = jnp.pad(dur_w, ((0, 0), (0, _KP - dur_w.shape[1])))
    b_pad = jnp.pad(dur_b, (0, _KP - dur_b.shape[0])).reshape(1, _KP)
    proj_table = _project_table(embed_table, w_pad, b_pad)
    ids2d = text_ids.reshape(-1, _IDXW)
    emb_flat, log_pad = _sc_gather(embed_table, proj_table, ids2d)
    embeddings = emb_flat.reshape(B, T, _D)
    logits = log_pad[:, : dur_b.shape[0]].reshape(B, T, dur_b.shape[0])
    return (embeddings, logits)


# speedup vs baseline: 2.9102x; 2.9102x over previous
import functools

import jax
import jax.numpy as jnp
from jax import lax
from jax.experimental import pallas as pl
from jax.experimental.pallas import tpu as pltpu
from jax.experimental.pallas import tpu_sc as plsc

_NC = 2
_NS = 16
_NW = _NC * _NS

_D = 64
_KP = 16

_CHUNK = 1024
_IDXW = 128
_NFIRE = _CHUNK // _IDXW


def _proj_body(tab_ref, w_ref, b_ref, out_ref):
    out_ref[...] = (
        jnp.dot(tab_ref[...], w_ref[...], preferred_element_type=jnp.float32)
        + b_ref[...]
    )


def _project_table(embed_table, w_pad, b_pad):
    V, D = embed_table.shape
    blk = 4000
    return pl.pallas_call(
        _proj_body,
        grid=(V // blk,),
        in_specs=[
            pl.BlockSpec((blk, D), lambda i: (i, 0)),
            pl.BlockSpec((D, _KP), lambda i: (0, 0)),
            pl.BlockSpec((1, _KP), lambda i: (0, 0)),
        ],
        out_specs=pl.BlockSpec((blk, _KP), lambda i: (i, 0)),
        out_shape=jax.ShapeDtypeStruct((V, _KP), jnp.float32),
    )(embed_table, w_pad, b_pad)


def _sc_gather(embed_table, proj_table, ids2d):
    n_rows = ids2d.shape[0] * ids2d.shape[1]
    per_w = n_rows // _NW
    n_chunks = per_w // _CHUNK
    mesh = plsc.VectorSubcoreMesh(
        core_axis_name="c", subcore_axis_name="s",
        num_cores=_NC, num_subcores=_NS,
    )

    @functools.partial(
        pl.kernel,
        mesh=mesh,
        out_type=(
            jax.ShapeDtypeStruct((n_rows, _D), jnp.float32),
            jax.ShapeDtypeStruct((n_rows, _KP), jnp.float32),
        ),
        scratch_types=[
            pltpu.VMEM((_NFIRE, _IDXW), jnp.int32),
            pltpu.VMEM((_CHUNK, _D), jnp.float32),
            pltpu.VMEM((_CHUNK, _KP), jnp.float32),
            pltpu.SemaphoreType.DMA,
        ],
        compiler_params=pltpu.CompilerParams(use_tc_tiling_on_sc=False),
    )
    def gather_kernel(tab_hbm, ptab_hbm, ids_hbm, emb_out, log_out,
                      idx_v, emb_v, log_v, sem):
        wid = lax.axis_index("s") * _NC + lax.axis_index("c")
        w_base = wid * per_w

        def body(i, _):
            base = pl.multiple_of(w_base + i * _CHUNK, _CHUNK)
            row0 = pl.multiple_of(base // _IDXW, _NFIRE)
            pltpu.sync_copy(ids_hbm.at[pl.ds(row0, _NFIRE)], idx_v)
            copies = []
            for j in range(_NFIRE):
                copies.append(pltpu.async_copy(
                    tab_hbm.at[idx_v.at[j]],
                    emb_v.at[pl.ds(j * _IDXW, _IDXW)], sem))
                copies.append(pltpu.async_copy(
                    ptab_hbm.at[idx_v.at[j]],
                    log_v.at[pl.ds(j * _IDXW, _IDXW)], sem))
            for c in copies:
                c.wait()
            pltpu.sync_copy(emb_v, emb_out.at[pl.ds(base, _CHUNK)])
            pltpu.sync_copy(log_v, log_out.at[pl.ds(base, _CHUNK)])
            return ()

        lax.fori_loop(0, n_chunks, body, ())

    return gather_kernel(embed_table, proj_table, ids2d)


def kernel(text_ids, embed_table, dur_w, dur_b):
    B, T = text_ids.shape
    w_pad = jnp.pad(dur_w, ((0, 0), (0, _KP - dur_w.shape[1])))
    b_pad = jnp.pad(dur_b, (0, _KP - dur_b.shape[0])).reshape(1, _KP)
    proj_table = _project_table(embed_table, w_pad, b_pad)
    ids2d = text_ids.reshape(-1, _IDXW)
    emb_flat, log_pad = _sc_gather(embed_table, proj_table, ids2d)
    embeddings = emb_flat.reshape(B, T, _D)
    logits = log_pad[:, : dur_b.shape[0]].reshape(B, T, dur_b.shape[0])
    return (embeddings, logits)


# speedup vs baseline: 7.5954x; 2.6099x over previous
import functools

import jax
import jax.numpy as jnp
from jax import lax
from jax.experimental import pallas as pl
from jax.experimental.pallas import tpu as pltpu
from jax.experimental.pallas import tpu_sc as plsc

_NC = 2
_NS = 16
_NW = _NC * _NS

_D = 64
_K = 10
_KP = 16

_B = 4096
_T = 200
_N = _B * _T

_GRP = 128
_GPC = 8
_CHUNK = _GRP * _GPC
_HALF = 512
_GBLK = 2 * _HALF


def _sc_gather(embed_table, ids2d):
    n_groups = _N // _GRP
    per_w = n_groups // _NW
    n_chunks = per_w // _GPC
    mesh = plsc.VectorSubcoreMesh(
        core_axis_name="c", subcore_axis_name="s",
        num_cores=_NC, num_subcores=_NS,
    )

    @functools.partial(
        pl.kernel,
        mesh=mesh,
        out_type=jax.ShapeDtypeStruct((_T, _B // _GBLK, _HALF, 2 * _D),
                                      jnp.float32),
        scratch_types=[
            pltpu.VMEM((_GPC, _GRP), jnp.int32),
            pltpu.VMEM((_CHUNK, _D), jnp.float32),
            pltpu.SemaphoreType.DMA,
        ],
        compiler_params=pltpu.CompilerParams(use_tc_tiling_on_sc=False),
    )
    def gather_kernel(tab_hbm, ids_hbm, out_hbm, idx_v, buf_v, sem):
        wid = lax.axis_index("s") * _NC + lax.axis_index("c")
        g_base = wid * per_w

        def body(c, _):
            g0 = pl.multiple_of(g_base + c * _GPC, _GPC)
            pltpu.sync_copy(ids_hbm.at[pl.ds(g0, _GPC)], idx_v)
            copies = [
                pltpu.async_copy(
                    tab_hbm.at[idx_v.at[j]],
                    buf_v.at[pl.ds(j * _GRP, _GRP)], sem)
                for j in range(_GPC)
            ]
            for cp in copies:
                cp.wait()
            for j in range(_GPC):
                n0 = (g0 + j) * _GRP
                t = n0 // _B
                b0 = n0 - t * _B
                g = b0 // _GBLK
                half = (b0 // _HALF) % 2
                r0 = pl.multiple_of(b0 % _HALF, _GRP)
                pltpu.sync_copy(
                    buf_v.at[pl.ds(j * _GRP, _GRP)],
                    out_hbm.at[t, g, pl.ds(r0, _GRP),
                               pl.ds(half * _D, _D)])
            return ()

        lax.fori_loop(0, n_chunks, body, ())

    return gather_kernel(embed_table, ids2d)


_TG = 8


def _tc_body(x_ref, wt_ref, b_ref, emb_ref, log_ref):
    for t in range(_TG):
        x = x_ref[t, 0]
        xt = jnp.transpose(x, (1, 0))
        e = xt[:_D, :]
        o = xt[_D:, :]
        emb_ref[t, :, 0:_HALF] = e
        emb_ref[t, :, _HALF:_GBLK] = o
        wt = wt_ref[...]
        bias = b_ref[...]
        le = jnp.dot(wt, e, preferred_element_type=jnp.float32) + bias
        lo = jnp.dot(wt, o, preferred_element_type=jnp.float32) + bias
        log_ref[:, t, 0:_HALF] = le[:_K, :]
        log_ref[:, t, _HALF:_GBLK] = lo[:_K, :]


def _tc_finish(staged, wt_pad, b_pad):
    n_gb = _B // _GBLK
    grid = (_T // _TG, n_gb)
    return pl.pallas_call(
        _tc_body,
        grid=grid,
        in_specs=[
            pl.BlockSpec((_TG, 1, _HALF, 2 * _D), lambda i, j: (i, j, 0, 0)),
            pl.BlockSpec((_KP, _D), lambda i, j: (0, 0)),
            pl.BlockSpec((_KP, 1), lambda i, j: (0, 0)),
        ],
        out_specs=[
            pl.BlockSpec((_TG, _D, _GBLK), lambda i, j: (i, 0, j)),
            pl.BlockSpec((_K, _TG, _GBLK), lambda i, j: (0, i, j)),
        ],
        out_shape=[
            jax.ShapeDtypeStruct((_T, _D, _B), jnp.float32),
            jax.ShapeDtypeStruct((_K, _T, _B), jnp.float32),
        ],
    )(staged, wt_pad, b_pad)


def kernel(text_ids, embed_table, dur_w, dur_b):
    ids2d = jnp.swapaxes(text_ids, 0, 1).reshape(_N // _GRP, _GRP)
    staged = _sc_gather(embed_table, ids2d)
    wt_pad = jnp.pad(jnp.transpose(dur_w), ((0, _KP - _K), (0, 0)))
    b_pad = jnp.pad(dur_b, (0, _KP - _K)).reshape(_KP, 1)
    emb_t, log_t = _tc_finish(staged, wt_pad, b_pad)
    embeddings = jnp.transpose(emb_t, (2, 0, 1))
    logits = jnp.transpose(log_t, (2, 1, 0))
    return (embeddings, logits)


# speedup vs baseline: 8.1345x; 1.0710x over previous
import functools

import jax
import jax.numpy as jnp
from jax import lax
from jax.experimental import pallas as pl
from jax.experimental.pallas import tpu as pltpu
from jax.experimental.pallas import tpu_sc as plsc

_NC = 2
_NS = 16
_NW = _NC * _NS

_D = 64
_K = 10
_KP = 16

_B = 4096
_T = 200
_N = _B * _T

_GRP = 128
_GPC = 8
_CHUNK = _GRP * _GPC
_HALF = 512
_GBLK = 2 * _HALF

_NSLICE = 5
_TS = _T // _NSLICE
_TG = 8


def _sc_gather_slice(embed_table, ids2d, t0):
    n_groups = _TS * (_B // _GRP)
    per_w = n_groups // _NW
    n_chunks = per_w // _GPC
    mesh = plsc.VectorSubcoreMesh(
        core_axis_name="c", subcore_axis_name="s",
        num_cores=_NC, num_subcores=_NS,
    )

    @functools.partial(
        pl.kernel,
        mesh=mesh,
        out_type=jax.ShapeDtypeStruct((_TS, _B // _GBLK, _HALF, 2 * _D),
                                      jnp.float32),
        scratch_types=[
            pltpu.VMEM((_GPC, _GRP), jnp.int32),
            pltpu.VMEM((_CHUNK, _D), jnp.float32),
            pltpu.SemaphoreType.DMA,
        ],
        compiler_params=pltpu.CompilerParams(use_tc_tiling_on_sc=False),
    )
    def gather_kernel(tab_hbm, ids_hbm, out_hbm, idx_v, buf_v, sem):
        wid = lax.axis_index("s") * _NC + lax.axis_index("c")
        g_base = wid * per_w

        def body(c, _):
            g0 = pl.multiple_of(g_base + c * _GPC, _GPC)
            pltpu.sync_copy(
                ids_hbm.at[pl.ds(t0 * (_B // _GRP) + g0, _GPC)], idx_v)
            copies = [
                pltpu.async_copy(
                    tab_hbm.at[idx_v.at[j]],
                    buf_v.at[pl.ds(j * _GRP, _GRP)], sem)
                for j in range(_GPC)
            ]
            for cp in copies:
                cp.wait()
            for j in range(_GPC):
                n0 = (g0 + j) * _GRP
                t = n0 // _B
                b0 = n0 - t * _B
                g = b0 // _GBLK
                half = (b0 // _HALF) % 2
                r0 = pl.multiple_of(b0 % _HALF, _GRP)
                pltpu.sync_copy(
                    buf_v.at[pl.ds(j * _GRP, _GRP)],
                    out_hbm.at[t, g, pl.ds(r0, _GRP),
                               pl.ds(half * _D, _D)])
            return ()

        lax.fori_loop(0, n_chunks, body, ())

    return gather_kernel(embed_table, ids2d)


def _tc_body(x_ref, wt_ref, b_ref, *rest):
    emb_ref, log_ref = rest[-2], rest[-1]
    for t in range(_TG):
        x = x_ref[t, 0]
        xt = jnp.transpose(x, (1, 0))
        e = xt[:_D, :]
        o = xt[_D:, :]
        emb_ref[t, :, 0:_HALF] = e
        emb_ref[t, :, _HALF:_GBLK] = o
        wt = wt_ref[...]
        bias = b_ref[...]
        le = jnp.dot(wt, e, preferred_element_type=jnp.float32) + bias
        lo = jnp.dot(wt, o, preferred_element_type=jnp.float32) + bias
        log_ref[:, t, 0:_HALF] = le[:_K, :]
        log_ref[:, t, _HALF:_GBLK] = lo[:_K, :]


def _tc_finish_slice(staged, wt_pad, b_pad, t0, prev):
    n_gb = _B // _GBLK
    grid = (_TS // _TG, n_gb)
    tb0 = t0 // _TG
    in_specs = [
        pl.BlockSpec((_TG, 1, _HALF, 2 * _D), lambda i, j: (i, j, 0, 0)),
        pl.BlockSpec((_KP, _D), lambda i, j: (0, 0)),
        pl.BlockSpec((_KP, 1), lambda i, j: (0, 0)),
    ]
    out_specs = [
        pl.BlockSpec((_TG, _D, _GBLK), lambda i, j: (tb0 + i, 0, j)),
        pl.BlockSpec((_K, _TG, _GBLK), lambda i, j: (0, tb0 + i, j)),
    ]
    out_shape = [
        jax.ShapeDtypeStruct((_T, _D, _B), jnp.float32),
        jax.ShapeDtypeStruct((_K, _T, _B), jnp.float32),
    ]
    args = [staged, wt_pad, b_pad]
    kwargs = {}
    if prev is not None:
        in_specs += [pl.BlockSpec(memory_space=pl.ANY),
                     pl.BlockSpec(memory_space=pl.ANY)]
        args += [prev[0], prev[1]]
        kwargs["input_output_aliases"] = {3: 0, 4: 1}
    return pl.pallas_call(
        _tc_body,
        grid=grid,
        in_specs=in_specs,
        out_specs=out_specs,
        out_shape=out_shape,
        **kwargs,
    )(*args)


def kernel(text_ids, embed_table, dur_w, dur_b):
    ids2d = jnp.swapaxes(text_ids, 0, 1).reshape(_N // _GRP, _GRP)
    wt_pad = jnp.pad(jnp.transpose(dur_w), ((0, _KP - _K), (0, 0)))
    b_pad = jnp.pad(dur_b, (0, _KP - _K)).reshape(_KP, 1)
    staged = [_sc_gather_slice(embed_table, ids2d, s * _TS)
              for s in range(_NSLICE)]
    prev = None
    for s in range(_NSLICE):
        prev = _tc_finish_slice(staged[s], wt_pad, b_pad, s * _TS, prev)
    emb_t, log_t = prev
    embeddings = jnp.transpose(emb_t, (2, 0, 1))
    logits = jnp.transpose(log_t, (2, 1, 0))
    return (embeddings, logits)
